# baseline (speedup 1.0000x reference)
import functools

import jax
import jax.numpy as jnp
from jax import lax
from jax.experimental import pallas as pl
from jax.experimental.pallas import tpu as pltpu
from jax.experimental.pallas import tpu_sc as plsc

NUM_GRAPHS = 256
N_NODES = 10000
N_EDGES = 320000
D_FEAT = 128

NC = 2
NS = 16
NW = NC * NS
E_W = N_EDGES // NW
C = 80
NR = 5
NG = E_W // (C * NR)
NCHUNK = NG * NR
HBM_SLOTS = (2,)

_mesh = plsc.VectorSubcoreMesh(core_axis_name="c", subcore_axis_name="s")


@functools.partial(
    pl.kernel,
    mesh=_mesh,
    out_type=jax.ShapeDtypeStruct((N_EDGES, D_FEAT), jnp.float32),
    scratch_types=[
        pltpu.VMEM_SHARED((NUM_GRAPHS, D_FEAT), jnp.float32),
        pltpu.VMEM_SHARED((N_NODES,), jnp.int32),
        pltpu.VMEM((E_W,), jnp.int32),
        pltpu.VMEM((E_W,), jnp.int32),
        pltpu.VMEM((NR, C, D_FEAT), jnp.float32),
        pltpu.SemaphoreType.DMA,
    ] + [pltpu.SemaphoreType.DMA] * (3 * NR),
)
def _unpool(gf_hbm, batch_hbm, esrc_hbm, out_hbm,
            gf_sh, batch_sh, idx_full, eb_full, rows, sem_st, *sems):
    sem_a = sems[:NR]
    sem_gr = sems[NR:2 * NR]
    sem_o = sems[2 * NR:]
    sid = lax.axis_index("s")
    wid = sid * NC + lax.axis_index("c")
    base = wid * E_W

    @pl.when(sid == 0)
    def _stage():
        pltpu.async_copy(gf_hbm, gf_sh, sem_st)
        pltpu.async_copy(batch_hbm, batch_sh, sem_st)

    pltpu.async_copy(esrc_hbm.at[pl.ds(base, E_W)], idx_full, sem_o[0])

    @pl.when(sid == 0)
    def _stage_wait():
        pltpu.make_async_copy(gf_hbm, gf_sh, sem_st).wait()
        pltpu.make_async_copy(batch_hbm, batch_sh, sem_st).wait()

    pltpu.make_async_copy(esrc_hbm.at[pl.ds(base, E_W)], idx_full,
                          sem_o[0]).wait()
    plsc.subcore_barrier()

    def fire_a(c, r):
        lo = c * C
        pltpu.async_copy(batch_sh.at[idx_full.at[pl.ds(lo, C)]],
                         eb_full.at[pl.ds(lo, C)], sem_a[r])

    def drain_a(c, r):
        lo = c * C
        pltpu.make_async_copy(batch_sh.at[idx_full.at[pl.ds(lo, C)]],
                              eb_full.at[pl.ds(lo, C)], sem_a[r]).wait()

    def _gsrc(r):
        return gf_hbm if r in HBM_SLOTS else gf_sh

    def fire_g(c, r):
        lo = c * C
        pltpu.async_copy(_gsrc(r).at[eb_full.at[pl.ds(lo, C)]],
                         rows.at[r], sem_gr[r])

    def drain_g(c, r):
        lo = c * C
        pltpu.make_async_copy(_gsrc(r).at[eb_full.at[pl.ds(lo, C)]],
                              rows.at[r], sem_gr[r]).wait()

    def fire_o(c, r):
        off = base + c * C
        pltpu.async_copy(rows.at[r], out_hbm.at[pl.ds(off, C)], sem_o[r])

    def drain_o(c, r):
        off = base + c * C
        pltpu.make_async_copy(rows.at[r], out_hbm.at[pl.ds(off, C)],
                              sem_o[r]).wait()

    for r in range(NR):
        fire_a(r, r)
    for r in range(NR):
        drain_a(r, r)
        fire_a(r + NR, r)
        fire_g(r, r)
        if r >= 3:
            drain_g(r - 3, r - 3)
            fire_o(r - 3, r - 3)

    def body(g, carry):
        for r in range(NR):
            c = g * NR + r
            drain_a(c, r)
            fire_a(c + NR, r)
            drain_o(c - NR, r)
            fire_g(c, r)
            drain_g(c - 3, (r - 3) % NR)
            fire_o(c - 3, (r - 3) % NR)
        return carry

    lax.fori_loop(1, NG - 1, body, 0)

    for r in range(NR):
        c = (NG - 1) * NR + r
        drain_a(c, r)
        drain_o(c - NR, r)
        fire_g(c, r)
        drain_g(c - 3, (r - 3) % NR)
        fire_o(c - 3, (r - 3) % NR)
    for c in (NCHUNK - 3, NCHUNK - 2, NCHUNK - 1):
        drain_g(c, c % NR)
        fire_o(c, c % NR)
    for r in range(NR):
        drain_o(NCHUNK - NR + r, r)


def kernel(graph_feat, batch, edge_index):
    edge_src = edge_index[0]
    return _unpool(graph_feat, batch, edge_src)


# speedup vs baseline: 1.6094x; 1.6094x over previous
import functools

import jax
import jax.numpy as jnp
from jax import lax
from jax.experimental import pallas as pl
from jax.experimental.pallas import tpu as pltpu
from jax.experimental.pallas import tpu_sc as plsc

NUM_GRAPHS = 256
N_NODES = 10000
N_EDGES = 320000
D_FEAT = 128

NC = 2
NS = 16
NW = NC * NS
E_W = N_EDGES // NW
C = 80
NR = 10
NA = 5
SK = 5
NCHUNK = E_W // C
PRO = 15
NBODY = (NCHUNK - PRO - 10) // NR
EPI = PRO + NBODY * NR

_mesh = plsc.VectorSubcoreMesh(core_axis_name="c", subcore_axis_name="s")


@functools.partial(
    pl.kernel,
    mesh=_mesh,
    out_type=jax.ShapeDtypeStruct((N_EDGES, D_FEAT), jnp.float32),
    scratch_types=[
        pltpu.VMEM_SHARED((NUM_GRAPHS, D_FEAT), jnp.float32),
        pltpu.VMEM_SHARED((N_NODES,), jnp.int32),
        pltpu.VMEM((E_W,), jnp.int32),
        pltpu.VMEM((E_W,), jnp.int32),
        pltpu.VMEM((NR, C, D_FEAT), jnp.float32),
        pltpu.SemaphoreType.DMA,
    ] + [pltpu.SemaphoreType.DMA] * (NA + 2 * NR),
)
def _unpool(gf_hbm, batch_hbm, esrc_hbm, out_hbm,
            gf_sh, batch_sh, idx_full, eb_full, rows, sem_st, *sems):
    sem_a = sems[:NA]
    sem_gr = sems[NA:NA + NR]
    sem_o = sems[NA + NR:]
    sid = lax.axis_index("s")
    wid = sid * NC + lax.axis_index("c")
    base = wid * E_W

    @pl.when(sid == 0)
    def _stage():
        pltpu.async_copy(gf_hbm, gf_sh, sem_st)
        pltpu.async_copy(batch_hbm, batch_sh, sem_st)

    pltpu.async_copy(esrc_hbm.at[pl.ds(base, E_W)], idx_full, sem_o[0])

    @pl.when(sid == 0)
    def _stage_wait():
        pltpu.make_async_copy(gf_hbm, gf_sh, sem_st).wait()
        pltpu.make_async_copy(batch_hbm, batch_sh, sem_st).wait()

    pltpu.make_async_copy(esrc_hbm.at[pl.ds(base, E_W)], idx_full,
                          sem_o[0]).wait()
    plsc.subcore_barrier()

    def fire_a(c, r):
        lo = c * C
        pltpu.async_copy(batch_sh.at[idx_full.at[pl.ds(lo, C)]],
                         eb_full.at[pl.ds(lo, C)], sem_a[r])

    def drain_a(c, r):
        lo = c * C
        pltpu.make_async_copy(batch_sh.at[idx_full.at[pl.ds(lo, C)]],
                              eb_full.at[pl.ds(lo, C)], sem_a[r]).wait()

    def fire_g(c, r):
        lo = c * C
        pltpu.async_copy(gf_sh.at[eb_full.at[pl.ds(lo, C)]],
                         rows.at[r], sem_gr[r])

    def drain_g(c, r):
        lo = c * C
        pltpu.make_async_copy(gf_sh.at[eb_full.at[pl.ds(lo, C)]],
                              rows.at[r], sem_gr[r]).wait()

    def fire_o(c, r):
        off = base + c * C
        pltpu.async_copy(rows.at[r], out_hbm.at[pl.ds(off, C)], sem_o[r])

    def drain_o(c, r):
        off = base + c * C
        pltpu.make_async_copy(rows.at[r], out_hbm.at[pl.ds(off, C)],
                              sem_o[r]).wait()

    for c in range(NA):
        fire_a(c, c % NA)
    for c in range(PRO):
        drain_a(c, c % NA)
        fire_a(c + NA, c % NA)
        if c >= NR:
            drain_o(c - NR, (c - NR) % NR)
        fire_g(c, c % NR)
        if c >= SK:
            drain_g(c - SK, (c - SK) % NR)
            fire_o(c - SK, (c - SK) % NR)

    def body(t, carry):
        for j in range(NR):
            c = PRO + t * NR + j
            drain_a(c, j % NA)
            fire_a(c + NA, j % NA)
            drain_o(c - NR, (PRO + j) % NR)
            fire_g(c, (PRO + j) % NR)
            drain_g(c - SK, j % NR)
            fire_o(c - SK, j % NR)
        return carry

    lax.fori_loop(0, NBODY, body, 0)

    for c in range(EPI, NCHUNK):
        drain_a(c, c % NA)
        if c + NA < NCHUNK:
            fire_a(c + NA, c % NA)
        drain_o(c - NR, (c - NR) % NR)
        fire_g(c, c % NR)
        drain_g(c - SK, (c - SK) % NR)
        fire_o(c - SK, (c - SK) % NR)
    for c in range(NCHUNK - SK, NCHUNK):
        drain_g(c, c % NR)
        fire_o(c, c % NR)
    for c in range(NCHUNK - NR, NCHUNK):
        drain_o(c, c % NR)


def kernel(graph_feat, batch, edge_index):
    edge_src = edge_index[0]
    return _unpool(graph_feat, batch, edge_src)


# speedup vs baseline: 1.6218x; 1.0077x over previous
import functools

import jax
import jax.numpy as jnp
from jax import lax
from jax.experimental import pallas as pl
from jax.experimental.pallas import tpu as pltpu
from jax.experimental.pallas import tpu_sc as plsc

NUM_GRAPHS = 256
N_NODES = 10000
N_EDGES = 320000
D_FEAT = 128

NC = 2
NS = 16
NW = NC * NS
E_W = N_EDGES // NW
C = 80
NR = 5
NG = E_W // (C * NR)
NCHUNK = NG * NR

_mesh = plsc.VectorSubcoreMesh(core_axis_name="c", subcore_axis_name="s")


@functools.partial(
    pl.kernel,
    mesh=_mesh,
    out_type=jax.ShapeDtypeStruct((N_EDGES, D_FEAT), jnp.float32),
    scratch_types=[
        pltpu.VMEM_SHARED((NUM_GRAPHS, D_FEAT), jnp.float32),
        pltpu.VMEM_SHARED((N_NODES,), jnp.int32),
        pltpu.VMEM((E_W,), jnp.int32),
        pltpu.VMEM((E_W,), jnp.int32),
        pltpu.VMEM((NR, C, D_FEAT), jnp.float32),
        pltpu.SemaphoreType.DMA,
    ] + [pltpu.SemaphoreType.DMA] * (3 * NR),
)
def _unpool(gf_hbm, batch_hbm, esrc_hbm, out_hbm,
            gf_sh, batch_sh, idx_full, eb_full, rows, sem_st, *sems):
    sem_a = sems[:NR]
    sem_gr = sems[NR:2 * NR]
    sem_o = sems[2 * NR:]
    sid = lax.axis_index("s")
    wid = sid * NC + lax.axis_index("c")
    base = wid * E_W

    @pl.when(sid == 0)
    def _stage():
        pltpu.async_copy(gf_hbm, gf_sh, sem_st)
        pltpu.async_copy(batch_hbm, batch_sh, sem_st)

    pltpu.async_copy(esrc_hbm.at[pl.ds(base, E_W)], idx_full, sem_o[0])

    @pl.when(sid == 0)
    def _stage_wait():
        pltpu.make_async_copy(gf_hbm, gf_sh, sem_st).wait()
        pltpu.make_async_copy(batch_hbm, batch_sh, sem_st).wait()

    pltpu.make_async_copy(esrc_hbm.at[pl.ds(base, E_W)], idx_full,
                          sem_o[0]).wait()
    plsc.subcore_barrier()

    def fire_a(c, r):
        lo = c * C
        pltpu.async_copy(batch_sh.at[idx_full.at[pl.ds(lo, C)]],
                         eb_full.at[pl.ds(lo, C)], sem_a[r])

    def drain_a(c, r):
        lo = c * C
        pltpu.make_async_copy(batch_sh.at[idx_full.at[pl.ds(lo, C)]],
                              eb_full.at[pl.ds(lo, C)], sem_a[r]).wait()

    def fire_g(c, r):
        lo = c * C
        pltpu.async_copy(gf_sh.at[eb_full.at[pl.ds(lo, C)]],
                         rows.at[r], sem_gr[r])

    def drain_g(c, r):
        lo = c * C
        pltpu.make_async_copy(gf_sh.at[eb_full.at[pl.ds(lo, C)]],
                              rows.at[r], sem_gr[r]).wait()

    def fire_o(c, r):
        off = base + c * C
        pltpu.async_copy(rows.at[r], out_hbm.at[pl.ds(off, C)], sem_o[r])

    def drain_o(c, r):
        off = base + c * C
        pltpu.make_async_copy(rows.at[r], out_hbm.at[pl.ds(off, C)],
                              sem_o[r]).wait()

    for r in range(NR):
        fire_a(r, r)
    for r in range(NR):
        drain_a(r, r)
        fire_a(r + NR, r)
        fire_g(r, r)
        if r >= 3:
            drain_g(r - 3, r - 3)
            fire_o(r - 3, r - 3)

    def body(g, carry):
        for r in range(NR):
            c = g * NR + r
            drain_a(c, r)
            fire_a(c + NR, r)
            drain_o(c - NR, r)
            fire_g(c, r)
            drain_g(c - 3, (r - 3) % NR)
            fire_o(c - 3, (r - 3) % NR)
        return carry

    lax.fori_loop(1, NG - 1, body, 0)

    for r in range(NR):
        c = (NG - 1) * NR + r
        drain_a(c, r)
        drain_o(c - NR, r)
        fire_g(c, r)
        drain_g(c - 3, (r - 3) % NR)
        fire_o(c - 3, (r - 3) % NR)
    for c in (NCHUNK - 3, NCHUNK - 2, NCHUNK - 1):
        drain_g(c, c % NR)
        fire_o(c, c % NR)
    for r in range(NR):
        drain_o(NCHUNK - NR + r, r)


def kernel(graph_feat, batch, edge_index):
    edge_src = edge_index[0]
    return _unpool(graph_feat, batch, edge_src)
